# Initial kernel scaffold; baseline (speedup 1.0000x reference)
import jax
import jax.numpy as jnp
from jax.experimental import pallas as pl


def kernel(x_nchw, w1, b1, w2, b2, w3, b3, wl1, bl1, wl2, bl2):
    raise NotImplementedError("write your pallas kernel here")


# speedup vs baseline: 1.6306x; 1.6306x over previous
import jax
import jax.numpy as jnp
from jax.experimental import pallas as pl
from jax.experimental.pallas import tpu as pltpu

_F32 = jnp.float32
_BF16 = jnp.bfloat16


def _band_matrices(w_oihw, S):
    w = w_oihw.astype(_F32)
    Cout, Cin = w.shape[0], w.shape[1]
    w_out = jnp.arange(S)
    w_in = jnp.arange(S)
    kw = w_in[:, None] - w_out[None, :] + 2
    valid = (kw >= 0) & (kw < 5)
    mats = []
    for kh in range(5):
        tap = jnp.transpose(w[:, :, kh, :], (2, 1, 0))
        band = jnp.where(valid[:, :, None, None],
                         tap[jnp.clip(kw, 0, 4)], 0.0)
        band = jnp.transpose(band, (0, 2, 1, 3))
        band = band.reshape(S, Cin, S // 2, 2, Cout)
        band = jnp.transpose(band, (0, 1, 3, 2, 4))
        mats.append(band.reshape(S * Cin, S * Cout))
    return jnp.stack(mats, axis=0)


def _pooled_bias(b, S):
    return jnp.tile(b.astype(_F32), S // 2).reshape(1, (S // 2) * b.shape[0])


def _cnn_kernel(x_ref, wb1_ref, bb1_ref, wb2_ref, bb2_ref, wb3_ref, bb3_ref,
                wl1_ref, bl1_ref, wl2_ref, bl2_ref,
                out_ref, pad2_ref, pad3_ref):
    B = x_ref.shape[1]

    lhs1 = jnp.concatenate(
        [x_ref[kh:kh + 32].reshape(32 * B, 128) for kh in range(5)], axis=1)
    a1 = jnp.dot(lhs1, wb1_ref[...], preferred_element_type=_F32)
    a1 = a1.reshape(16, 2, B, 1024)
    a1 = jnp.maximum(a1[:, 0], a1[:, 1])
    p1 = jnp.maximum(a1[..., :512], a1[..., 512:]) + bb1_ref[...]
    pad2_ref[0:2] = jnp.zeros((2, B, 512), _BF16)
    pad2_ref[2:18] = p1.astype(_BF16)
    pad2_ref[18:20] = jnp.zeros((2, B, 512), _BF16)

    a2 = jnp.dot(pad2_ref[0:16].reshape(16 * B, 512), wb2_ref[0],
                 preferred_element_type=_F32)
    for kh in range(1, 5):
        a2 = a2 + jnp.dot(pad2_ref[kh:kh + 16].reshape(16 * B, 512),
                          wb2_ref[kh], preferred_element_type=_F32)
    a2 = a2.reshape(8, 2, B, 512)
    a2 = jnp.maximum(a2[:, 0], a2[:, 1])
    p2 = jnp.maximum(a2[..., :256], a2[..., 256:]) + bb2_ref[...]
    pad3_ref[0:2] = jnp.zeros((2, B, 256), _BF16)
    pad3_ref[2:10] = p2.astype(_BF16)
    pad3_ref[10:12] = jnp.zeros((2, B, 256), _BF16)

    a3 = jnp.dot(pad3_ref[0:8].reshape(8 * B, 256), wb3_ref[0],
                 preferred_element_type=_F32)
    for kh in range(1, 5):
        a3 = a3 + jnp.dot(pad3_ref[kh:kh + 8].reshape(8 * B, 256),
                          wb3_ref[kh], preferred_element_type=_F32)
    a3 = a3.reshape(4, 2, B, 512)
    a3 = jnp.maximum(a3[:, 0], a3[:, 1])
    p3 = jnp.maximum(a3[..., :256], a3[..., 256:]) + bb3_ref[...]

    h1 = jnp.dot(p3[0], wl1_ref[0], preferred_element_type=_F32)
    for h in range(1, 4):
        h1 = h1 + jnp.dot(p3[h], wl1_ref[h], preferred_element_type=_F32)
    h1 = h1 + bl1_ref[...]
    out_ref[...] = (jnp.dot(h1, wl2_ref[...], preferred_element_type=_F32)
                    + bl2_ref[...])


def _forward(x_nchw, params, *, block_b=64):
    N = x_nchw.shape[0]
    B = block_b
    n_pad = (-N) % B
    Npad = N + n_pad

    x = jnp.transpose(x_nchw.astype(_BF16), (2, 0, 3, 1)).reshape(32, N, 96)
    x = jnp.pad(x, ((2, 2), (0, n_pad), (0, 32)))

    wb1 = _band_matrices(params["w1"], 32)
    wb1 = jnp.pad(wb1, ((0, 0), (0, 32), (0, 0)))
    wb1 = wb1.reshape(640, 1024).astype(_BF16)
    wb2 = _band_matrices(params["w2"], 16).astype(_BF16)
    wb3 = _band_matrices(params["w3"], 8).astype(_BF16)
    bb1 = _pooled_bias(params["b1"], 32)
    bb2 = _pooled_bias(params["b2"], 16)
    bb3 = _pooled_bias(params["b3"], 8)

    wl1r = jnp.transpose(params["wl1"].astype(_F32).reshape(64, 64, 4, 4),
                         (2, 3, 1, 0)).reshape(4, 256, 64)
    bl1 = params["bl1"].astype(_F32).reshape(1, 64)
    wl2p = jnp.zeros((64, 128), _F32).at[:, :10].set(params["wl2"].astype(_F32).T)
    bl2p = jnp.zeros((1, 128), _F32).at[:, :10].set(
        params["bl2"].astype(_F32).reshape(1, 10))

    grid_spec = pltpu.PrefetchScalarGridSpec(
        num_scalar_prefetch=0,
        grid=(Npad // B,),
        in_specs=[
            pl.BlockSpec((36, B, 128), lambda i: (0, i, 0)),
            pl.BlockSpec((640, 1024), lambda i: (0, 0)),
            pl.BlockSpec((1, 512), lambda i: (0, 0)),
            pl.BlockSpec((5, 512, 512), lambda i: (0, 0, 0)),
            pl.BlockSpec((1, 256), lambda i: (0, 0)),
            pl.BlockSpec((5, 256, 512), lambda i: (0, 0, 0)),
            pl.BlockSpec((1, 256), lambda i: (0, 0)),
            pl.BlockSpec((4, 256, 64), lambda i: (0, 0, 0)),
            pl.BlockSpec((1, 64), lambda i: (0, 0)),
            pl.BlockSpec((64, 128), lambda i: (0, 0)),
            pl.BlockSpec((1, 128), lambda i: (0, 0)),
        ],
        out_specs=pl.BlockSpec((B, 128), lambda i: (i, 0)),
        scratch_shapes=[
            pltpu.VMEM((20, B, 512), _BF16),
            pltpu.VMEM((12, B, 256), _BF16),
        ],
    )

    out = pl.pallas_call(
        _cnn_kernel,
        out_shape=jax.ShapeDtypeStruct((Npad, 128), _F32),
        grid_spec=grid_spec,
        compiler_params=pltpu.CompilerParams(
            dimension_semantics=("parallel",),
            vmem_limit_bytes=60 * 1024 * 1024),
    )(x, wb1, bb1, wb2, bb2, wb3, bb3, wl1r, bl1, wl2p, bl2p)

    return out[:N, :10]


def kernel(x_nchw, w1, b1, w2, b2, w3, b3, wl1, bl1, wl2, bl2):
    params = {"w1": w1, "b1": b1, "w2": w2, "b2": b2, "w3": w3, "b3": b3,
              "wl1": wl1, "bl1": bl1, "wl2": wl2, "bl2": bl2}
    return _forward(x_nchw, params)


# speedup vs baseline: 1.7123x; 1.0501x over previous
import jax
import jax.numpy as jnp
from jax.experimental import pallas as pl
from jax.experimental.pallas import tpu as pltpu

_F32 = jnp.float32
_BF16 = jnp.bfloat16


def _band_matrices(w_oihw, S, cw_rows=False):
    w = w_oihw.astype(_F32)
    Cout, Cin = w.shape[0], w.shape[1]
    kw = jnp.arange(S)[:, None] - jnp.arange(S)[None, :] + 2
    valid = (kw >= 0) & (kw < 5)
    mats = []
    for kh in range(5):
        tap = jnp.transpose(w[:, :, kh, :], (2, 1, 0))
        band = jnp.where(valid[:, :, None, None],
                         tap[jnp.clip(kw, 0, 4)], 0.0)
        if cw_rows:
            band = jnp.transpose(band, (2, 0, 1, 3))
        else:
            band = jnp.transpose(band, (0, 2, 1, 3))
        band = band.reshape(S * Cin, S // 2, 2, Cout)
        band = jnp.transpose(band, (0, 2, 1, 3))
        mats.append(band.reshape(S * Cin, S * Cout))
    return jnp.stack(mats, axis=0)


def _pooled_bias(b, S):
    return jnp.tile(b.astype(_F32), S // 2).reshape(1, (S // 2) * b.shape[0])


def _cnn_kernel(x_ref, wb1_ref, bb1_ref, wb2_ref, bb2_ref, wb3_ref, bb3_ref,
                wl1_ref, bl1_ref, wl2_ref, bl2_ref,
                out_ref, xs_ref, l1_ref, l2_ref, l3_ref, pad2_ref, pad3_ref):
    B = x_ref.shape[1]

    xs_ref[0:2] = jnp.zeros((2, B, 128), _BF16)
    xs_ref[2:34, :, 0:96] = x_ref[...]
    xs_ref[2:34, :, 96:128] = jnp.zeros((32, B, 32), _BF16)
    xs_ref[34:36] = jnp.zeros((2, B, 128), _BF16)

    for kh in range(5):
        l1_ref[:, 128 * kh:128 * (kh + 1)] = xs_ref[kh:kh + 32].reshape(32 * B, 128)
    a1 = jnp.dot(l1_ref[...], wb1_ref[...], preferred_element_type=_F32)
    a1 = a1.reshape(16, 2, B, 1024)
    a1 = jnp.maximum(a1[:, 0], a1[:, 1])
    p1 = jnp.maximum(a1[..., :512], a1[..., 512:]) + bb1_ref[...]
    pad2_ref[0:2] = jnp.zeros((2, B, 512), _BF16)
    pad2_ref[2:18] = p1.astype(_BF16)
    pad2_ref[18:20] = jnp.zeros((2, B, 512), _BF16)

    for kh in range(5):
        l2_ref[:, 512 * kh:512 * (kh + 1)] = pad2_ref[kh:kh + 16].reshape(16 * B, 512)
    a2 = jnp.dot(l2_ref[...], wb2_ref[...], preferred_element_type=_F32)
    a2 = a2.reshape(8, 2, B, 512)
    a2 = jnp.maximum(a2[:, 0], a2[:, 1])
    p2 = jnp.maximum(a2[..., :256], a2[..., 256:]) + bb2_ref[...]
    pad3_ref[0:2] = jnp.zeros((2, B, 256), _BF16)
    pad3_ref[2:10] = p2.astype(_BF16)
    pad3_ref[10:12] = jnp.zeros((2, B, 256), _BF16)

    for kh in range(5):
        l3_ref[:, 256 * kh:256 * (kh + 1)] = pad3_ref[kh:kh + 8].reshape(8 * B, 256)
    a3 = jnp.dot(l3_ref[...], wb3_ref[...], preferred_element_type=_F32)
    a3 = a3.reshape(4, 2, B, 512)
    a3 = jnp.maximum(a3[:, 0], a3[:, 1])
    p3 = jnp.maximum(a3[..., :256], a3[..., 256:]) + bb3_ref[...]

    h1 = jnp.dot(p3[0], wl1_ref[0], preferred_element_type=_F32)
    for h in range(1, 4):
        h1 = h1 + jnp.dot(p3[h], wl1_ref[h], preferred_element_type=_F32)
    h1 = h1 + bl1_ref[...]
    out_ref[...] = (jnp.dot(h1, wl2_ref[...], preferred_element_type=_F32)
                    + bl2_ref[...])


def _forward(x_nchw, params, *, block_b=64):
    N = x_nchw.shape[0]
    B = block_b
    n_pad = (-N) % B
    Npad = N + n_pad

    x = jnp.transpose(x_nchw.astype(_BF16), (2, 0, 1, 3)).reshape(32, N, 96)
    if n_pad:
        x = jnp.pad(x, ((0, 0), (0, n_pad), (0, 0)))

    wb1 = _band_matrices(params["w1"], 32, cw_rows=True)
    wb1 = jnp.pad(wb1, ((0, 0), (0, 32), (0, 0)))
    wb1 = wb1.reshape(640, 1024).astype(_BF16)
    wb2 = _band_matrices(params["w2"], 16)
    wb2 = wb2.reshape(2560, 512).astype(_BF16)
    wb3 = _band_matrices(params["w3"], 8)
    wb3 = wb3.reshape(1280, 512).astype(_BF16)
    bb1 = _pooled_bias(params["b1"], 32)
    bb2 = _pooled_bias(params["b2"], 16)
    bb3 = _pooled_bias(params["b3"], 8)

    wl1r = jnp.transpose(params["wl1"].astype(_F32).reshape(64, 64, 4, 4),
                         (2, 3, 1, 0)).reshape(4, 256, 64)
    bl1 = params["bl1"].astype(_F32).reshape(1, 64)
    wl2p = jnp.zeros((64, 128), _F32).at[:, :10].set(params["wl2"].astype(_F32).T)
    bl2p = jnp.zeros((1, 128), _F32).at[:, :10].set(
        params["bl2"].astype(_F32).reshape(1, 10))

    grid_spec = pltpu.PrefetchScalarGridSpec(
        num_scalar_prefetch=0,
        grid=(Npad // B,),
        in_specs=[
            pl.BlockSpec((32, B, 96), lambda i: (0, i, 0)),
            pl.BlockSpec((640, 1024), lambda i: (0, 0)),
            pl.BlockSpec((1, 512), lambda i: (0, 0)),
            pl.BlockSpec((2560, 512), lambda i: (0, 0)),
            pl.BlockSpec((1, 256), lambda i: (0, 0)),
            pl.BlockSpec((1280, 512), lambda i: (0, 0)),
            pl.BlockSpec((1, 256), lambda i: (0, 0)),
            pl.BlockSpec((4, 256, 64), lambda i: (0, 0, 0)),
            pl.BlockSpec((1, 64), lambda i: (0, 0)),
            pl.BlockSpec((64, 128), lambda i: (0, 0)),
            pl.BlockSpec((1, 128), lambda i: (0, 0)),
        ],
        out_specs=pl.BlockSpec((B, 128), lambda i: (i, 0)),
        scratch_shapes=[
            pltpu.VMEM((36, B, 128), _BF16),
            pltpu.VMEM((32 * B, 640), _BF16),
            pltpu.VMEM((16 * B, 2560), _BF16),
            pltpu.VMEM((8 * B, 1280), _BF16),
            pltpu.VMEM((20, B, 512), _BF16),
            pltpu.VMEM((12, B, 256), _BF16),
        ],
    )

    out = pl.pallas_call(
        _cnn_kernel,
        out_shape=jax.ShapeDtypeStruct((Npad, 128), _F32),
        grid_spec=grid_spec,
        compiler_params=pltpu.CompilerParams(
            dimension_semantics=("parallel",),
            vmem_limit_bytes=60 * 1024 * 1024),
    )(x, wb1, bb1, wb2, bb2, wb3, bb3, wl1r, bl1, wl2p, bl2p)

    return out[:N, :10]


def kernel(x_nchw, w1, b1, w2, b2, w3, b3, wl1, bl1, wl2, bl2):
    params = {"w1": w1, "b1": b1, "w2": w2, "b2": b2, "w3": w3, "b3": b3,
              "wl1": wl1, "bl1": bl1, "wl2": wl2, "bl2": bl2}
    return _forward(x_nchw, params)


# speedup vs baseline: 1.9887x; 1.1614x over previous
import jax
import jax.numpy as jnp
from jax.experimental import pallas as pl
from jax.experimental.pallas import tpu as pltpu

_F32 = jnp.float32
_BF16 = jnp.bfloat16


def _band_matrices(w_oihw, S, cw_rows=False):
    w = w_oihw.astype(_F32)
    Cout, Cin = w.shape[0], w.shape[1]
    kw = jnp.arange(S)[:, None] - jnp.arange(S)[None, :] + 2
    valid = (kw >= 0) & (kw < 5)
    mats = []
    for kh in range(5):
        tap = jnp.transpose(w[:, :, kh, :], (2, 1, 0))
        band = jnp.where(valid[:, :, None, None],
                         tap[jnp.clip(kw, 0, 4)], 0.0)
        if cw_rows:
            band = jnp.transpose(band, (2, 0, 1, 3))
        else:
            band = jnp.transpose(band, (0, 2, 1, 3))
        band = band.reshape(S * Cin, S // 2, 2, Cout)
        band = jnp.transpose(band, (0, 2, 1, 3))
        mats.append(band.reshape(S * Cin, S * Cout))
    return jnp.stack(mats, axis=0)


def _pooled_bias(b, S):
    return jnp.tile(b.astype(_F32), S // 2).reshape(1, (S // 2) * b.shape[0])


def _cnn_kernel(x_ref, wb1_ref, bb1_ref, w2a_ref, w2b_ref, bb2_ref,
                wb3_ref, bb3_ref, wl1_ref, bl1_ref, wl2_ref, bl2_ref,
                out_ref, xs_ref, l1_ref, l2a_ref, l2b_ref, l3_ref,
                pad2_ref, pad3_ref):
    B = x_ref.shape[1]

    xs_ref[0:2] = jnp.zeros((2, B, 96), _BF16)
    xs_ref[2:34] = x_ref[...]
    xs_ref[34:36] = jnp.zeros((2, B, 96), _BF16)

    for kh in range(5):
        l1_ref[:, 96 * kh:96 * (kh + 1)] = xs_ref[kh:kh + 32].reshape(32 * B, 96)
    l1_ref[:, 480:512] = jnp.zeros((32 * B, 32), _BF16)
    a1 = jnp.dot(l1_ref[...], wb1_ref[...], preferred_element_type=_F32)
    a1 = a1.reshape(16, 2, B, 1024)
    a1 = jnp.maximum(a1[:, 0], a1[:, 1])
    p1 = jnp.maximum(a1[..., :512], a1[..., 512:]) + bb1_ref[...]
    pad2_ref[0:2] = jnp.zeros((2, B, 512), _BF16)
    pad2_ref[2:18] = p1.astype(_BF16)
    pad2_ref[18:20] = jnp.zeros((2, B, 512), _BF16)

    for kh in range(5):
        l2a_ref[:, 320 * kh:320 * (kh + 1)] = (
            pad2_ref[kh:kh + 16, :, 0:320].reshape(16 * B, 320))
        l2b_ref[:, 320 * kh:320 * (kh + 1)] = (
            pad2_ref[kh:kh + 16, :, 192:512].reshape(16 * B, 320))
    l2a_ref[:, 1600:1792] = jnp.zeros((16 * B, 192), _BF16)
    l2b_ref[:, 1600:1792] = jnp.zeros((16 * B, 192), _BF16)
    a2a = jnp.dot(l2a_ref[...], w2a_ref[...], preferred_element_type=_F32)
    a2b = jnp.dot(l2b_ref[...], w2b_ref[...], preferred_element_type=_F32)
    a2a = a2a.reshape(8, 2, B, 256)
    a2b = a2b.reshape(8, 2, B, 256)
    a2a = jnp.maximum(a2a[:, 0], a2a[:, 1])
    a2b = jnp.maximum(a2b[:, 0], a2b[:, 1])
    p2a = jnp.maximum(a2a[..., :128], a2a[..., 128:])
    p2b = jnp.maximum(a2b[..., :128], a2b[..., 128:])
    p2 = jnp.concatenate([p2a, p2b], axis=2) + bb2_ref[...]
    pad3_ref[0:2] = jnp.zeros((2, B, 256), _BF16)
    pad3_ref[2:10] = p2.astype(_BF16)
    pad3_ref[10:12] = jnp.zeros((2, B, 256), _BF16)

    for kh in range(5):
        l3_ref[:, 256 * kh:256 * (kh + 1)] = pad3_ref[kh:kh + 8].reshape(8 * B, 256)
    a3 = jnp.dot(l3_ref[...], wb3_ref[...], preferred_element_type=_F32)
    a3 = a3.reshape(4, 2, B, 512)
    a3 = jnp.maximum(a3[:, 0], a3[:, 1])
    p3 = jnp.maximum(a3[..., :256], a3[..., 256:]) + bb3_ref[...]

    h1 = jnp.dot(p3[0], wl1_ref[0], preferred_element_type=_F32)
    for h in range(1, 4):
        h1 = h1 + jnp.dot(p3[h], wl1_ref[h], preferred_element_type=_F32)
    h1 = h1 + bl1_ref[...]
    out_ref[...] = (jnp.dot(h1, wl2_ref[...], preferred_element_type=_F32)
                    + bl2_ref[...])


def _forward(x_nchw, params, *, block_b=64):
    N = x_nchw.shape[0]
    B = block_b
    n_pad = (-N) % B
    Npad = N + n_pad

    x = jnp.transpose(x_nchw.astype(_BF16), (2, 0, 1, 3)).reshape(32, N, 96)
    if n_pad:
        x = jnp.pad(x, ((0, 0), (0, n_pad), (0, 0)))

    wb1 = jnp.pad(
        _band_matrices(params["w1"], 32, cw_rows=True).reshape(480, 1024),
        ((0, 32), (0, 0))).astype(_BF16)

    wb2 = _band_matrices(params["w2"], 16)
    w2a = jnp.concatenate([wb2[:, 0:320, 0:128], wb2[:, 0:320, 256:384]],
                          axis=2).reshape(1600, 256)
    w2b = jnp.concatenate([wb2[:, 192:512, 128:256], wb2[:, 192:512, 384:512]],
                          axis=2).reshape(1600, 256)
    w2a = jnp.pad(w2a, ((0, 192), (0, 0))).astype(_BF16)
    w2b = jnp.pad(w2b, ((0, 192), (0, 0))).astype(_BF16)

    wb3 = _band_matrices(params["w3"], 8)
    wb3 = wb3.reshape(1280, 512).astype(_BF16)
    bb1 = _pooled_bias(params["b1"], 32)
    bb2 = _pooled_bias(params["b2"], 16)
    bb3 = _pooled_bias(params["b3"], 8)

    wl1r = jnp.transpose(params["wl1"].astype(_F32).reshape(64, 64, 4, 4),
                         (2, 3, 1, 0)).reshape(4, 256, 64)
    bl1 = params["bl1"].astype(_F32).reshape(1, 64)
    wl2p = jnp.zeros((64, 16), _F32).at[:, :10].set(params["wl2"].astype(_F32).T)
    bl2p = jnp.zeros((1, 16), _F32).at[:, :10].set(
        params["bl2"].astype(_F32).reshape(1, 10))

    grid_spec = pltpu.PrefetchScalarGridSpec(
        num_scalar_prefetch=0,
        grid=(Npad // B,),
        in_specs=[
            pl.BlockSpec((32, B, 96), lambda i: (0, i, 0)),
            pl.BlockSpec((512, 1024), lambda i: (0, 0)),
            pl.BlockSpec((1, 512), lambda i: (0, 0)),
            pl.BlockSpec((1792, 256), lambda i: (0, 0)),
            pl.BlockSpec((1792, 256), lambda i: (0, 0)),
            pl.BlockSpec((1, 256), lambda i: (0, 0)),
            pl.BlockSpec((1280, 512), lambda i: (0, 0)),
            pl.BlockSpec((1, 256), lambda i: (0, 0)),
            pl.BlockSpec((4, 256, 64), lambda i: (0, 0, 0)),
            pl.BlockSpec((1, 64), lambda i: (0, 0)),
            pl.BlockSpec((64, 16), lambda i: (0, 0)),
            pl.BlockSpec((1, 16), lambda i: (0, 0)),
        ],
        out_specs=pl.BlockSpec((B, 16), lambda i: (i, 0)),
        scratch_shapes=[
            pltpu.VMEM((36, B, 96), _BF16),
            pltpu.VMEM((32 * B, 512), _BF16),
            pltpu.VMEM((16 * B, 1792), _BF16),
            pltpu.VMEM((16 * B, 1792), _BF16),
            pltpu.VMEM((8 * B, 1280), _BF16),
            pltpu.VMEM((20, B, 512), _BF16),
            pltpu.VMEM((12, B, 256), _BF16),
        ],
    )

    out = pl.pallas_call(
        _cnn_kernel,
        out_shape=jax.ShapeDtypeStruct((Npad, 16), _F32),
        grid_spec=grid_spec,
        compiler_params=pltpu.CompilerParams(
            dimension_semantics=("parallel",),
            vmem_limit_bytes=60 * 1024 * 1024),
    )(x, wb1, bb1, w2a, w2b, bb2, wb3, bb3, wl1r, bl1, wl2p, bl2p)

    return out[:N, :10]


def kernel(x_nchw, w1, b1, w2, b2, w3, b3, wl1, bl1, wl2, bl2):
    params = {"w1": w1, "b1": b1, "w2": w2, "b2": b2, "w3": w3, "b3": b3,
              "wl1": wl1, "bl1": bl1, "wl2": wl2, "bl2": bl2}
    return _forward(x_nchw, params)


# speedup vs baseline: 2.0691x; 1.0404x over previous
import jax
import jax.numpy as jnp
from jax.experimental import pallas as pl
from jax.experimental.pallas import tpu as pltpu

_F32 = jnp.float32
_BF16 = jnp.bfloat16


def _band_matrices(w_oihw, S, cw_rows=False):
    w = w_oihw.astype(_F32)
    Cout, Cin = w.shape[0], w.shape[1]
    diff = jnp.arange(S)[:, None] - jnp.arange(S)[None, :] + 2
    sel = (diff[:, :, None] == jnp.arange(5)).astype(_F32)
    spec = 'iok,dchk->hciod' if cw_rows else 'iok,dchk->hicod'
    band = jnp.einsum(spec, sel, w)
    band = band.reshape(5, S * Cin, S // 2, 2, Cout)
    band = jnp.transpose(band, (0, 1, 3, 2, 4))
    return band.reshape(5, S * Cin, S * Cout)


def _pooled_bias(b, S):
    return jnp.tile(b.astype(_F32), S // 2).reshape(1, (S // 2) * b.shape[0])


def _cnn_kernel(x_ref, wb1_ref, bb1_ref, w2a_ref, w2b_ref, bb2_ref,
                wb3_ref, bb3_ref, wl1_ref, bl1_ref, wl2_ref, bl2_ref,
                out_ref, xs_ref, l1_ref, l2a_ref, l2b_ref, l3_ref,
                pad2_ref, pad3_ref):
    B = x_ref.shape[0]

    xs_ref[0:2] = jnp.zeros((2, B, 96), _BF16)
    for c in range(3):
        xs_ref[2:34, :, 32 * c:32 * (c + 1)] = jnp.transpose(
            x_ref[:, 32 * c:32 * (c + 1), :], (1, 0, 2)).astype(_BF16)
    xs_ref[34:36] = jnp.zeros((2, B, 96), _BF16)

    for kh in range(5):
        l1_ref[:, 96 * kh:96 * (kh + 1)] = xs_ref[kh:kh + 32].reshape(32 * B, 96)
    l1_ref[:, 480:512] = jnp.zeros((32 * B, 32), _BF16)
    a1 = jnp.dot(l1_ref[...], wb1_ref[...], preferred_element_type=_F32)
    a1 = a1.reshape(16, 2, B, 1024)
    a1 = jnp.maximum(a1[:, 0], a1[:, 1])
    p1 = jnp.maximum(a1[..., :512], a1[..., 512:]) + bb1_ref[...]
    pad2_ref[0:2] = jnp.zeros((2, B, 512), _BF16)
    pad2_ref[2:18] = p1.astype(_BF16)
    pad2_ref[18:20] = jnp.zeros((2, B, 512), _BF16)

    for kh in range(5):
        l2a_ref[:, 320 * kh:320 * (kh + 1)] = (
            pad2_ref[kh:kh + 16, :, 0:320].reshape(16 * B, 320))
        l2b_ref[:, 320 * kh:320 * (kh + 1)] = (
            pad2_ref[kh:kh + 16, :, 192:512].reshape(16 * B, 320))
    l2a_ref[:, 1600:1792] = jnp.zeros((16 * B, 192), _BF16)
    l2b_ref[:, 1600:1792] = jnp.zeros((16 * B, 192), _BF16)
    a2a = jnp.dot(l2a_ref[...], w2a_ref[...], preferred_element_type=_F32)
    a2b = jnp.dot(l2b_ref[...], w2b_ref[...], preferred_element_type=_F32)
    a2a = a2a.reshape(8, 2, B, 256)
    a2b = a2b.reshape(8, 2, B, 256)
    a2a = jnp.maximum(a2a[:, 0], a2a[:, 1])
    a2b = jnp.maximum(a2b[:, 0], a2b[:, 1])
    p2a = jnp.maximum(a2a[..., :128], a2a[..., 128:])
    p2b = jnp.maximum(a2b[..., :128], a2b[..., 128:])
    p2 = jnp.concatenate([p2a, p2b], axis=2) + bb2_ref[...]
    pad3_ref[0:2] = jnp.zeros((2, B, 256), _BF16)
    pad3_ref[2:10] = p2.astype(_BF16)
    pad3_ref[10:12] = jnp.zeros((2, B, 256), _BF16)

    for kh in range(5):
        l3_ref[:, 256 * kh:256 * (kh + 1)] = pad3_ref[kh:kh + 8].reshape(8 * B, 256)
    a3 = jnp.dot(l3_ref[...], wb3_ref[...], preferred_element_type=_F32)
    a3 = a3.reshape(4, 2, B, 512)
    a3 = jnp.maximum(a3[:, 0], a3[:, 1])
    p3 = jnp.maximum(a3[..., :256], a3[..., 256:]) + bb3_ref[...]

    fc_lhs = jnp.concatenate([p3[0], p3[1], p3[2], p3[3]], axis=1)
    h1 = jnp.dot(fc_lhs, wl1_ref[...], preferred_element_type=_F32)
    h1 = h1 + bl1_ref[...]
    out_ref[...] = (jnp.dot(h1, wl2_ref[...], preferred_element_type=_F32)
                    + bl2_ref[...])


def _forward(x_nchw, params, *, block_b=64):
    N = x_nchw.shape[0]
    B = block_b
    n_pad = (-N) % B
    Npad = N + n_pad

    x = x_nchw.reshape(N, 96, 32)
    if n_pad:
        x = jnp.pad(x, ((0, n_pad), (0, 0), (0, 0)))

    wb1 = jnp.pad(
        _band_matrices(params["w1"], 32, cw_rows=True).reshape(480, 1024),
        ((0, 32), (0, 0))).astype(_BF16)

    wb2 = _band_matrices(params["w2"], 16)
    w2a = jnp.concatenate([wb2[:, 0:320, 0:128], wb2[:, 0:320, 256:384]],
                          axis=2).reshape(1600, 256)
    w2b = jnp.concatenate([wb2[:, 192:512, 128:256], wb2[:, 192:512, 384:512]],
                          axis=2).reshape(1600, 256)
    w2a = jnp.pad(w2a, ((0, 192), (0, 0))).astype(_BF16)
    w2b = jnp.pad(w2b, ((0, 192), (0, 0))).astype(_BF16)

    wb3 = _band_matrices(params["w3"], 8)
    wb3 = wb3.reshape(1280, 512).astype(_BF16)
    bb1 = _pooled_bias(params["b1"], 32)
    bb2 = _pooled_bias(params["b2"], 16)
    bb3 = _pooled_bias(params["b3"], 8)

    wl1r = jnp.transpose(params["wl1"].astype(_F32).reshape(64, 64, 4, 4),
                         (2, 3, 1, 0)).reshape(1024, 64)
    bl1 = params["bl1"].astype(_F32).reshape(1, 64)
    wl2p = jnp.zeros((64, 16), _F32).at[:, :10].set(params["wl2"].astype(_F32).T)
    bl2p = jnp.zeros((1, 16), _F32).at[:, :10].set(
        params["bl2"].astype(_F32).reshape(1, 10))

    grid_spec = pltpu.PrefetchScalarGridSpec(
        num_scalar_prefetch=0,
        grid=(Npad // B,),
        in_specs=[
            pl.BlockSpec((B, 96, 32), lambda i: (i, 0, 0)),
            pl.BlockSpec((512, 1024), lambda i: (0, 0)),
            pl.BlockSpec((1, 512), lambda i: (0, 0)),
            pl.BlockSpec((1792, 256), lambda i: (0, 0)),
            pl.BlockSpec((1792, 256), lambda i: (0, 0)),
            pl.BlockSpec((1, 256), lambda i: (0, 0)),
            pl.BlockSpec((1280, 512), lambda i: (0, 0)),
            pl.BlockSpec((1, 256), lambda i: (0, 0)),
            pl.BlockSpec((1024, 64), lambda i: (0, 0)),
            pl.BlockSpec((1, 64), lambda i: (0, 0)),
            pl.BlockSpec((64, 16), lambda i: (0, 0)),
            pl.BlockSpec((1, 16), lambda i: (0, 0)),
        ],
        out_specs=pl.BlockSpec((B, 16), lambda i: (i, 0)),
        scratch_shapes=[
            pltpu.VMEM((36, B, 96), _BF16),
            pltpu.VMEM((32 * B, 512), _BF16),
            pltpu.VMEM((16 * B, 1792), _BF16),
            pltpu.VMEM((16 * B, 1792), _BF16),
            pltpu.VMEM((8 * B, 1280), _BF16),
            pltpu.VMEM((20, B, 512), _BF16),
            pltpu.VMEM((12, B, 256), _BF16),
        ],
    )

    out = pl.pallas_call(
        _cnn_kernel,
        out_shape=jax.ShapeDtypeStruct((Npad, 16), _F32),
        grid_spec=grid_spec,
        compiler_params=pltpu.CompilerParams(
            dimension_semantics=("parallel",),
            vmem_limit_bytes=60 * 1024 * 1024),
    )(x, wb1, bb1, w2a, w2b, bb2, wb3, bb3, wl1r, bl1, wl2p, bl2p)

    return out[:N, :10]


def kernel(x_nchw, w1, b1, w2, b2, w3, b3, wl1, bl1, wl2, bl2):
    params = {"w1": w1, "b1": b1, "w2": w2, "b2": b2, "w3": w3, "b3": b3,
              "wl1": wl1, "bl1": bl1, "wl2": wl2, "bl2": bl2}
    return _forward(x_nchw, params)


# speedup vs baseline: 2.1048x; 1.0173x over previous
import jax
import jax.numpy as jnp
from jax.experimental import pallas as pl
from jax.experimental.pallas import tpu as pltpu

_F32 = jnp.float32
_BF16 = jnp.bfloat16


def _band_matrices(w_oihw, S, cw_rows=False):
    w = w_oihw.astype(_F32)
    Cout, Cin = w.shape[0], w.shape[1]
    diff = jnp.arange(S)[:, None] - jnp.arange(S)[None, :] + 2
    sel = (diff[:, :, None] == jnp.arange(5)).astype(_F32)
    spec = 'iok,dchk->hciod' if cw_rows else 'iok,dchk->hicod'
    band = jnp.einsum(spec, sel, w)
    band = band.reshape(5, S * Cin, S // 2, 2, Cout)
    band = jnp.transpose(band, (0, 1, 3, 2, 4))
    return band.reshape(5, S * Cin, S * Cout)


def _pooled_bias(b, S):
    return jnp.tile(b.astype(_F32), S // 2).reshape(1, (S // 2) * b.shape[0])


def _cnn_kernel(x_ref, wb1_ref, bb1_ref, w2a_ref, w2b_ref, bb2_ref,
                w3a_ref, w3b_ref, bb3_ref, wl1_ref, bl1_ref, wl2_ref, bl2_ref,
                out_ref, xs_ref, l1_ref, l2a_ref, l2b_ref, l3a_ref, l3b_ref,
                pad2_ref, pad3_ref):
    B = x_ref.shape[0]

    xs_ref[0:2] = jnp.zeros((2, B, 96), _BF16)
    for c in range(3):
        xs_ref[2:34, :, 32 * c:32 * (c + 1)] = jnp.transpose(
            x_ref[:, 32 * c:32 * (c + 1), :].astype(_BF16), (1, 0, 2))
    xs_ref[34:36] = jnp.zeros((2, B, 96), _BF16)

    for kh in range(5):
        l1_ref[:, 96 * kh:96 * (kh + 1)] = xs_ref[kh:kh + 32].reshape(32 * B, 96)
    l1_ref[:, 480:512] = jnp.zeros((32 * B, 32), _BF16)
    a1 = jnp.dot(l1_ref[...], wb1_ref[...], preferred_element_type=_F32)
    a1 = a1.reshape(16, 2, B, 1024)
    a1 = jnp.maximum(a1[:, 0], a1[:, 1])
    p1 = jnp.maximum(a1[..., :512], a1[..., 512:]) + bb1_ref[...]
    pad2_ref[0:2] = jnp.zeros((2, B, 512), _BF16)
    pad2_ref[2:18] = p1.astype(_BF16)
    pad2_ref[18:20] = jnp.zeros((2, B, 512), _BF16)

    for kh in range(5):
        l2a_ref[:, 320 * kh:320 * (kh + 1)] = (
            pad2_ref[kh:kh + 16, :, 0:320].reshape(16 * B, 320))
        l2b_ref[:, 320 * kh:320 * (kh + 1)] = (
            pad2_ref[kh:kh + 16, :, 192:512].reshape(16 * B, 320))
    l2a_ref[:, 1600:1792] = jnp.zeros((16 * B, 192), _BF16)
    l2b_ref[:, 1600:1792] = jnp.zeros((16 * B, 192), _BF16)
    a2a = jnp.dot(l2a_ref[...], w2a_ref[...], preferred_element_type=_F32)
    a2b = jnp.dot(l2b_ref[...], w2b_ref[...], preferred_element_type=_F32)
    a2a = a2a.reshape(8, 2, B, 256)
    a2b = a2b.reshape(8, 2, B, 256)
    a2a = jnp.maximum(a2a[:, 0], a2a[:, 1])
    a2b = jnp.maximum(a2b[:, 0], a2b[:, 1])
    p2a = jnp.maximum(a2a[..., :128], a2a[..., 128:])
    p2b = jnp.maximum(a2b[..., :128], a2b[..., 128:])
    p2 = jnp.concatenate([p2a, p2b], axis=2) + bb2_ref[...]
    pad3_ref[0:2] = jnp.zeros((2, B, 256), _BF16)
    pad3_ref[2:10] = p2.astype(_BF16)
    pad3_ref[10:12] = jnp.zeros((2, B, 256), _BF16)

    for kh in range(5):
        l3a_ref[:, 192 * kh:192 * (kh + 1)] = (
            pad3_ref[kh:kh + 8, :, 0:192].reshape(8 * B, 192))
        l3b_ref[:, 192 * kh:192 * (kh + 1)] = (
            pad3_ref[kh:kh + 8, :, 64:256].reshape(8 * B, 192))
    l3a_ref[:, 960:1024] = jnp.zeros((8 * B, 64), _BF16)
    l3b_ref[:, 960:1024] = jnp.zeros((8 * B, 64), _BF16)
    a3a = jnp.dot(l3a_ref[...], w3a_ref[...], preferred_element_type=_F32)
    a3b = jnp.dot(l3b_ref[...], w3b_ref[...], preferred_element_type=_F32)
    a3a = a3a.reshape(4, 2, B, 256)
    a3b = a3b.reshape(4, 2, B, 256)
    a3a = jnp.maximum(a3a[:, 0], a3a[:, 1])
    a3b = jnp.maximum(a3b[:, 0], a3b[:, 1])
    p3a = jnp.maximum(a3a[..., :128], a3a[..., 128:])
    p3b = jnp.maximum(a3b[..., :128], a3b[..., 128:])
    p3 = jnp.concatenate([p3a, p3b], axis=2) + bb3_ref[...]

    fc_lhs = jnp.concatenate([p3[0], p3[1], p3[2], p3[3]], axis=1)
    h1 = jnp.dot(fc_lhs, wl1_ref[...], preferred_element_type=_F32)
    h1 = h1 + bl1_ref[...]
    out_ref[...] = (jnp.dot(h1, wl2_ref[...], preferred_element_type=_F32)
                    + bl2_ref[...])


def _forward(x_nchw, params, *, block_b=64):
    N = x_nchw.shape[0]
    B = block_b
    n_pad = (-N) % B
    Npad = N + n_pad

    x = x_nchw.reshape(N, 96, 32)
    if n_pad:
        x = jnp.pad(x, ((0, n_pad), (0, 0), (0, 0)))

    wb1 = jnp.pad(
        _band_matrices(params["w1"], 32, cw_rows=True).reshape(480, 1024),
        ((0, 32), (0, 0))).astype(_BF16)

    wb2 = _band_matrices(params["w2"], 16)
    w2a = jnp.concatenate([wb2[:, 0:320, 0:128], wb2[:, 0:320, 256:384]],
                          axis=2).reshape(1600, 256)
    w2b = jnp.concatenate([wb2[:, 192:512, 128:256], wb2[:, 192:512, 384:512]],
                          axis=2).reshape(1600, 256)
    w2a = jnp.pad(w2a, ((0, 192), (0, 0))).astype(_BF16)
    w2b = jnp.pad(w2b, ((0, 192), (0, 0))).astype(_BF16)

    wb3 = _band_matrices(params["w3"], 8)
    w3a = jnp.concatenate([wb3[:, 0:192, 0:128], wb3[:, 0:192, 256:384]],
                          axis=2).reshape(960, 256)
    w3b = jnp.concatenate([wb3[:, 64:256, 128:256], wb3[:, 64:256, 384:512]],
                          axis=2).reshape(960, 256)
    w3a = jnp.pad(w3a, ((0, 64), (0, 0))).astype(_BF16)
    w3b = jnp.pad(w3b, ((0, 64), (0, 0))).astype(_BF16)
    bb1 = _pooled_bias(params["b1"], 32)
    bb2 = _pooled_bias(params["b2"], 16)
    bb3 = _pooled_bias(params["b3"], 8)

    wl1r = jnp.transpose(params["wl1"].astype(_F32).reshape(64, 64, 4, 4),
                         (2, 3, 1, 0)).reshape(1024, 64)
    bl1 = params["bl1"].astype(_F32).reshape(1, 64)
    wl2p = jnp.zeros((64, 16), _F32).at[:, :10].set(params["wl2"].astype(_F32).T)
    bl2p = jnp.zeros((1, 16), _F32).at[:, :10].set(
        params["bl2"].astype(_F32).reshape(1, 10))

    grid_spec = pltpu.PrefetchScalarGridSpec(
        num_scalar_prefetch=0,
        grid=(Npad // B,),
        in_specs=[
            pl.BlockSpec((B, 96, 32), lambda i: (i, 0, 0)),
            pl.BlockSpec((512, 1024), lambda i: (0, 0)),
            pl.BlockSpec((1, 512), lambda i: (0, 0)),
            pl.BlockSpec((1792, 256), lambda i: (0, 0)),
            pl.BlockSpec((1792, 256), lambda i: (0, 0)),
            pl.BlockSpec((1, 256), lambda i: (0, 0)),
            pl.BlockSpec((1024, 256), lambda i: (0, 0)),
            pl.BlockSpec((1024, 256), lambda i: (0, 0)),
            pl.BlockSpec((1, 256), lambda i: (0, 0)),
            pl.BlockSpec((1024, 64), lambda i: (0, 0)),
            pl.BlockSpec((1, 64), lambda i: (0, 0)),
            pl.BlockSpec((64, 16), lambda i: (0, 0)),
            pl.BlockSpec((1, 16), lambda i: (0, 0)),
        ],
        out_specs=pl.BlockSpec((B, 16), lambda i: (i, 0)),
        scratch_shapes=[
            pltpu.VMEM((36, B, 96), _BF16),
            pltpu.VMEM((32 * B, 512), _BF16),
            pltpu.VMEM((16 * B, 1792), _BF16),
            pltpu.VMEM((16 * B, 1792), _BF16),
            pltpu.VMEM((8 * B, 1024), _BF16),
            pltpu.VMEM((8 * B, 1024), _BF16),
            pltpu.VMEM((20, B, 512), _BF16),
            pltpu.VMEM((12, B, 256), _BF16),
        ],
    )

    out = pl.pallas_call(
        _cnn_kernel,
        out_shape=jax.ShapeDtypeStruct((Npad, 16), _F32),
        grid_spec=grid_spec,
        compiler_params=pltpu.CompilerParams(
            dimension_semantics=("parallel",),
            vmem_limit_bytes=60 * 1024 * 1024),
    )(x, wb1, bb1, w2a, w2b, bb2, w3a, w3b, bb3, wl1r, bl1, wl2p, bl2p)

    return out[:N, :10]


def kernel(x_nchw, w1, b1, w2, b2, w3, b3, wl1, bl1, wl2, bl2):
    params = {"w1": w1, "b1": b1, "w2": w2, "b2": b2, "w3": w3, "b3": b3,
              "wl1": wl1, "bl1": bl1, "wl2": wl2, "bl2": bl2}
    return _forward(x_nchw, params)


# speedup vs baseline: 2.1189x; 1.0067x over previous
import jax
import jax.numpy as jnp
from jax.experimental import pallas as pl
from jax.experimental.pallas import tpu as pltpu

_F32 = jnp.float32
_BF16 = jnp.bfloat16


def _band_matrices(w_oihw, S, cw_rows=False):
    w = w_oihw.astype(_F32)
    Cout, Cin = w.shape[0], w.shape[1]
    diff = jnp.arange(S)[:, None] - jnp.arange(S)[None, :] + 2
    sel = (diff[:, :, None] == jnp.arange(5)).astype(_F32)
    spec = 'iok,dchk->hciod' if cw_rows else 'iok,dchk->hicod'
    band = jnp.einsum(spec, sel, w)
    band = band.reshape(5, S * Cin, S // 2, 2, Cout)
    band = jnp.transpose(band, (0, 1, 3, 2, 4))
    return band.reshape(5, S * Cin, S * Cout)


def _pooled_bias(b, S):
    return jnp.tile(b.astype(_F32), S // 2).reshape(1, (S // 2) * b.shape[0])


def _cnn_kernel(x_ref, wb1_ref, bb1_ref, w2a_ref, w2b_ref, bb2_ref,
                w3a_ref, w3b_ref, bb3_ref, wl1_ref, bl1_ref, wl2_ref, bl2_ref,
                out_ref, xs_ref, l1_ref, l2a_ref, l2b_ref, l3a_ref, l3b_ref,
                pad2_ref, pad3_ref):
    B = x_ref.shape[0]

    xs_ref[0:2] = jnp.zeros((2, B, 96), _BF16)
    for c in range(3):
        xs_ref[2:34, :, 32 * c:32 * (c + 1)] = jnp.transpose(
            x_ref[:, 32 * c:32 * (c + 1), :].astype(_BF16), (1, 0, 2))
    xs_ref[34:36] = jnp.zeros((2, B, 96), _BF16)

    for kh in range(5):
        l1_ref[:, 96 * kh:96 * (kh + 1)] = xs_ref[kh:kh + 32].reshape(32 * B, 96)
    l1_ref[:, 480:512] = jnp.zeros((32 * B, 32), _BF16)
    a1 = jnp.dot(l1_ref[...], wb1_ref[...], preferred_element_type=_F32)
    a1 = a1.reshape(16, 2, B, 1024)
    a1 = jnp.maximum(a1[:, 0], a1[:, 1])
    p1 = jnp.maximum(a1[..., :512], a1[..., 512:]) + bb1_ref[...]
    pad2_ref[0:2] = jnp.zeros((2, B, 512), _BF16)
    pad2_ref[2:18] = p1.astype(_BF16)
    pad2_ref[18:20] = jnp.zeros((2, B, 512), _BF16)

    for kh in range(5):
        l2a_ref[:, 320 * kh:320 * (kh + 1)] = (
            pad2_ref[kh:kh + 16, :, 0:320].reshape(16 * B, 320))
        l2b_ref[:, 320 * kh:320 * (kh + 1)] = (
            pad2_ref[kh:kh + 16, :, 192:512].reshape(16 * B, 320))
    l2a_ref[:, 1600:1792] = jnp.zeros((16 * B, 192), _BF16)
    l2b_ref[:, 1600:1792] = jnp.zeros((16 * B, 192), _BF16)
    a2a = jnp.dot(l2a_ref[...], w2a_ref[...], preferred_element_type=_F32)
    a2b = jnp.dot(l2b_ref[...], w2b_ref[...], preferred_element_type=_F32)
    a2a = a2a.reshape(8, 2, B, 256)
    a2b = a2b.reshape(8, 2, B, 256)
    a2a = jnp.maximum(a2a[:, 0], a2a[:, 1])
    a2b = jnp.maximum(a2b[:, 0], a2b[:, 1])
    p2a = jnp.maximum(a2a[..., :128], a2a[..., 128:])
    p2b = jnp.maximum(a2b[..., :128], a2b[..., 128:])
    p2 = jnp.concatenate([p2a, p2b], axis=2) + bb2_ref[...]
    pad3_ref[0:2] = jnp.zeros((2, B, 256), _BF16)
    pad3_ref[2:10] = p2.astype(_BF16)
    pad3_ref[10:12] = jnp.zeros((2, B, 256), _BF16)

    for kh in range(5):
        l3a_ref[:, 192 * kh:192 * (kh + 1)] = (
            pad3_ref[kh:kh + 8, :, 0:192].reshape(8 * B, 192))
        l3b_ref[:, 192 * kh:192 * (kh + 1)] = (
            pad3_ref[kh:kh + 8, :, 64:256].reshape(8 * B, 192))
    l3a_ref[:, 960:1024] = jnp.zeros((8 * B, 64), _BF16)
    l3b_ref[:, 960:1024] = jnp.zeros((8 * B, 64), _BF16)
    a3a = jnp.dot(l3a_ref[...], w3a_ref[...], preferred_element_type=_F32)
    a3b = jnp.dot(l3b_ref[...], w3b_ref[...], preferred_element_type=_F32)
    a3a = a3a.reshape(4, 2, B, 256)
    a3b = a3b.reshape(4, 2, B, 256)
    a3a = jnp.maximum(a3a[:, 0], a3a[:, 1])
    a3b = jnp.maximum(a3b[:, 0], a3b[:, 1])
    p3a = jnp.maximum(a3a[..., :128], a3a[..., 128:])
    p3b = jnp.maximum(a3b[..., :128], a3b[..., 128:])
    p3 = jnp.concatenate([p3a, p3b], axis=2) + bb3_ref[...]

    fc_lhs = jnp.concatenate([p3[0], p3[1], p3[2], p3[3]], axis=1)
    h1 = jnp.dot(fc_lhs, wl1_ref[...], preferred_element_type=_F32)
    h1 = h1 + bl1_ref[...]
    out_ref[...] = (jnp.dot(h1, wl2_ref[...], preferred_element_type=_F32)
                    + bl2_ref[...])


def _forward(x_nchw, params, *, block_b=128):
    N = x_nchw.shape[0]
    B = block_b
    n_pad = (-N) % B
    Npad = N + n_pad

    x = x_nchw.reshape(N, 96, 32)
    if n_pad:
        x = jnp.pad(x, ((0, n_pad), (0, 0), (0, 0)))

    wb1 = jnp.pad(
        _band_matrices(params["w1"], 32, cw_rows=True).reshape(480, 1024),
        ((0, 32), (0, 0))).astype(_BF16)

    wb2 = _band_matrices(params["w2"], 16)
    w2a = jnp.concatenate([wb2[:, 0:320, 0:128], wb2[:, 0:320, 256:384]],
                          axis=2).reshape(1600, 256)
    w2b = jnp.concatenate([wb2[:, 192:512, 128:256], wb2[:, 192:512, 384:512]],
                          axis=2).reshape(1600, 256)
    w2a = jnp.pad(w2a, ((0, 192), (0, 0))).astype(_BF16)
    w2b = jnp.pad(w2b, ((0, 192), (0, 0))).astype(_BF16)

    wb3 = _band_matrices(params["w3"], 8)
    w3a = jnp.concatenate([wb3[:, 0:192, 0:128], wb3[:, 0:192, 256:384]],
                          axis=2).reshape(960, 256)
    w3b = jnp.concatenate([wb3[:, 64:256, 128:256], wb3[:, 64:256, 384:512]],
                          axis=2).reshape(960, 256)
    w3a = jnp.pad(w3a, ((0, 64), (0, 0))).astype(_BF16)
    w3b = jnp.pad(w3b, ((0, 64), (0, 0))).astype(_BF16)
    bb1 = _pooled_bias(params["b1"], 32)
    bb2 = _pooled_bias(params["b2"], 16)
    bb3 = _pooled_bias(params["b3"], 8)

    wl1r = jnp.transpose(params["wl1"].astype(_F32).reshape(64, 64, 4, 4),
                         (2, 3, 1, 0)).reshape(1024, 64)
    bl1 = params["bl1"].astype(_F32).reshape(1, 64)
    wl2p = jnp.zeros((64, 16), _F32).at[:, :10].set(params["wl2"].astype(_F32).T)
    bl2p = jnp.zeros((1, 16), _F32).at[:, :10].set(
        params["bl2"].astype(_F32).reshape(1, 10))

    grid_spec = pltpu.PrefetchScalarGridSpec(
        num_scalar_prefetch=0,
        grid=(Npad // B,),
        in_specs=[
            pl.BlockSpec((B, 96, 32), lambda i: (i, 0, 0)),
            pl.BlockSpec((512, 1024), lambda i: (0, 0)),
            pl.BlockSpec((1, 512), lambda i: (0, 0)),
            pl.BlockSpec((1792, 256), lambda i: (0, 0)),
            pl.BlockSpec((1792, 256), lambda i: (0, 0)),
            pl.BlockSpec((1, 256), lambda i: (0, 0)),
            pl.BlockSpec((1024, 256), lambda i: (0, 0)),
            pl.BlockSpec((1024, 256), lambda i: (0, 0)),
            pl.BlockSpec((1, 256), lambda i: (0, 0)),
            pl.BlockSpec((1024, 64), lambda i: (0, 0)),
            pl.BlockSpec((1, 64), lambda i: (0, 0)),
            pl.BlockSpec((64, 16), lambda i: (0, 0)),
            pl.BlockSpec((1, 16), lambda i: (0, 0)),
        ],
        out_specs=pl.BlockSpec((B, 16), lambda i: (i, 0)),
        scratch_shapes=[
            pltpu.VMEM((36, B, 96), _BF16),
            pltpu.VMEM((32 * B, 512), _BF16),
            pltpu.VMEM((16 * B, 1792), _BF16),
            pltpu.VMEM((16 * B, 1792), _BF16),
            pltpu.VMEM((8 * B, 1024), _BF16),
            pltpu.VMEM((8 * B, 1024), _BF16),
            pltpu.VMEM((20, B, 512), _BF16),
            pltpu.VMEM((12, B, 256), _BF16),
        ],
    )

    out = pl.pallas_call(
        _cnn_kernel,
        out_shape=jax.ShapeDtypeStruct((Npad, 16), _F32),
        grid_spec=grid_spec,
        compiler_params=pltpu.CompilerParams(
            dimension_semantics=("parallel",),
            vmem_limit_bytes=60 * 1024 * 1024),
    )(x, wb1, bb1, w2a, w2b, bb2, w3a, w3b, bb3, wl1r, bl1, wl2p, bl2p)

    return out[:N, :10]


def kernel(x_nchw, w1, b1, w2, b2, w3, b3, wl1, bl1, wl2, bl2):
    params = {"w1": w1, "b1": b1, "w2": w2, "b2": b2, "w3": w3, "b3": b3,
              "wl1": wl1, "bl1": bl1, "wl2": wl2, "bl2": bl2}
    return _forward(x_nchw, params)
